# baseline (speedup 1.0000x reference)
import functools

import jax
import jax.numpy as jnp
from jax import lax
from jax.experimental import pallas as pl
from jax.experimental.pallas import tpu as pltpu
from jax.experimental.pallas import tpu_sc as plsc

EPS = 1e-5
NW = 32


def _sc_gather_emb(x_pad, emb_table, nf):
    batch, np_ = x_pad.shape
    x_flat = x_pad.reshape(-1)
    d = emb_table.shape[1]
    per_w = batch // NW
    ns = 32
    n_ch = per_w // ns
    mesh = plsc.VectorSubcoreMesh(core_axis_name="c", subcore_axis_name="s")

    @functools.partial(
        pl.kernel,
        mesh=mesh,
        out_type=jax.ShapeDtypeStruct((nf * batch, d), jnp.float32),
        scratch_types=[
            pltpu.VMEM((ns * np_,), jnp.int32),
            pltpu.SemaphoreType.DMA,
        ],
    )
    def k(x_hbm, emb_hbm, out_hbm, x_v, sem):
        wid = lax.axis_index("s") * 2 + lax.axis_index("c")
        sbase = wid * per_w

        for c in range(n_ch):
            pltpu.sync_copy(
                x_hbm.at[pl.ds((sbase + c * ns) * np_, ns * np_)], x_v)
            cbase = sbase + c * ns

            def fire(s, _):
                v0 = x_v[pl.ds(s * np_, 16)]
                v1 = x_v[pl.ds(s * np_ + 16, 16)]
                for f in range(nf):
                    r = v0[f] if f < 16 else v1[f - 16]
                    pltpu.async_copy(
                        emb_hbm.at[pl.ds(r, 1), :],
                        out_hbm.at[pl.ds(f * batch + cbase + s, 1), :],
                        sem,
                    )
                return 0
            lax.fori_loop(0, ns, fire, 0)
            pltpu.make_async_copy(
                out_hbm.at[pl.ds(0, ns * nf)],
                out_hbm.at[pl.ds(0, ns * nf)], sem).wait()

    return k(x_flat, emb_table)


def _sc_gather_lin(x_flat, lin_flat):
    n = x_flat.shape[0]
    per_w = n // NW
    mesh = plsc.VectorSubcoreMesh(core_axis_name="c", subcore_axis_name="s")

    @functools.partial(
        pl.kernel,
        mesh=mesh,
        compiler_params=pltpu.CompilerParams(use_tc_tiling_on_sc=False),
        out_type=jax.ShapeDtypeStruct((n,), jnp.float32),
        scratch_types=[
            pltpu.VMEM((per_w,), jnp.int32),
            pltpu.VMEM((per_w,), jnp.float32),
            pltpu.SemaphoreType.DMA,
        ],
    )
    def k(x_hbm, lin_hbm, lin_out, idx_v, lin_v, sem):
        wid = lax.axis_index("s") * 2 + lax.axis_index("c")
        base = wid * per_w
        pltpu.sync_copy(x_hbm.at[pl.ds(base, per_w)], idx_v)
        pltpu.async_copy(lin_hbm.at[idx_v], lin_v, sem).wait()
        pltpu.sync_copy(lin_v, lin_out.at[pl.ds(base, per_w)])

    return k(x_flat, lin_flat)


def _fm_l1_body(e_ref, lin_ref, w1_ref, b1_ref,
                h1_ref, fm_ref, s_acc, q_acc, fm1_s):
    g = pl.program_id(1)
    ng = pl.num_programs(1)
    e_g = e_ref[...]
    hpart = jnp.dot(e_g, w1_ref[0], preferred_element_type=jnp.float32,
                    precision=lax.Precision.HIGHEST)
    qpart = jnp.sum(e_g * e_g, axis=1, keepdims=True)

    @pl.when(g == 0)
    def _():
        h1_ref[...] = hpart + b1_ref[...]
        s_acc[...] = e_g
        q_acc[...] = qpart
        fm1_s[...] = jnp.sum(lin_ref[...], axis=1, keepdims=True)

    @pl.when(g > 0)
    def _():
        h1_ref[...] += hpart
        s_acc[...] += e_g
        q_acc[...] += qpart

    @pl.when(g == ng - 1)
    def _():
        s = s_acc[...]
        fm_ref[...] = fm1_s[...] + 0.5 * (
            jnp.sum(s * s, axis=1, keepdims=True) - q_acc[...])


def _bn(h, g, be):
    mean = jnp.mean(h, axis=0, keepdims=True)
    var = jnp.mean((h - mean) ** 2, axis=0, keepdims=True)
    return (h - mean) * lax.rsqrt(var + EPS) * g + be


def _head_body(h1_ref, fm_ref, w2_ref, b2_ref, w3_ref, b3_ref, w4_ref, b4_ref,
               g1_ref, be1_ref, g2_ref, be2_ref, g3_ref, be3_ref, out_ref):
    h = _bn(h1_ref[...], g1_ref[...], be1_ref[...])
    h = jnp.maximum(h, 0.0)
    h = jnp.dot(h, w2_ref[...], preferred_element_type=jnp.float32,
                precision=lax.Precision.HIGHEST) + b2_ref[...]
    h = _bn(h, g2_ref[...], be2_ref[...])
    h = jnp.maximum(h, 0.0)
    h = jnp.dot(h, w3_ref[...], preferred_element_type=jnp.float32,
                precision=lax.Precision.HIGHEST) + b3_ref[...]
    h = _bn(h, g3_ref[...], be3_ref[...])
    h = jnp.maximum(h, 0.0)
    deep = jnp.dot(h, w4_ref[...], preferred_element_type=jnp.float32,
                   precision=lax.Precision.HIGHEST) + b4_ref[...]
    out_ref[...] = jax.nn.sigmoid(fm_ref[...] + deep)


def kernel(x, emb_table, lin_table, W1, b1, W2, b2, W3, b3, W4, b4,
           g1, be1, g2, be2, g3, be3):
    batch, nf = x.shape
    d = emb_table.shape[1]
    npair = nf // 2
    h1_dim = W1.shape[1]
    x_flat = x.reshape(-1)

    x_pad = jnp.pad(x, ((0, 0), (0, 32 - nf)))
    ef = _sc_gather_emb(x_pad, emb_table, nf)
    lin_rows = _sc_gather_lin(x_flat, lin_table.reshape(-1))
    lin = lin_rows.reshape(batch, nf)

    w1_k = W1.reshape(nf, d, h1_dim)

    bb = 512
    grid_i = batch // bb
    h1, fm = pl.pallas_call(
        _fm_l1_body,
        grid=(grid_i, nf),
        in_specs=[
            pl.BlockSpec((bb, d), lambda i, g: (g * grid_i + i, 0)),
            pl.BlockSpec((bb, nf), lambda i, g: (i, 0)),
            pl.BlockSpec((1, d, h1_dim), lambda i, g: (g, 0, 0)),
            pl.BlockSpec((1, h1_dim), lambda i, g: (0, 0)),
        ],
        out_specs=[
            pl.BlockSpec((bb, h1_dim), lambda i, g: (i, 0)),
            pl.BlockSpec((bb, 1), lambda i, g: (i, 0)),
        ],
        out_shape=[
            jax.ShapeDtypeStruct((batch, h1_dim), jnp.float32),
            jax.ShapeDtypeStruct((batch, 1), jnp.float32),
        ],
        scratch_shapes=[
            pltpu.VMEM((bb, d), jnp.float32),
            pltpu.VMEM((bb, 1), jnp.float32),
            pltpu.VMEM((bb, 1), jnp.float32),
        ],
    )(ef, lin, w1_k, b1.reshape(1, -1))

    row = lambda v: v.reshape(1, -1)
    out = pl.pallas_call(
        _head_body,
        out_shape=jax.ShapeDtypeStruct((batch, 1), jnp.float32),
    )(h1, fm, W2, row(b2), W3, row(b3), W4, row(b4),
      row(g1), row(be1), row(g2), row(be2), row(g3), row(be3))
    return out


# speedup vs baseline: 3.2812x; 3.2812x over previous
import functools

import jax
import jax.numpy as jnp
from jax import lax
from jax.experimental import pallas as pl
from jax.experimental.pallas import tpu as pltpu
from jax.experimental.pallas import tpu_sc as plsc

EPS = 1e-5
NW = 32


def _sc_gather(x_flat, emb_table, lin_table):
    n = x_flat.shape[0]
    d = emb_table.shape[1]
    per_w = n // NW
    ch = 416
    n_ch = per_w // ch
    lin2 = lin_table.reshape(-1)
    mesh = plsc.VectorSubcoreMesh(core_axis_name="c", subcore_axis_name="s")

    @functools.partial(
        pl.kernel,
        mesh=mesh,
        compiler_params=pltpu.CompilerParams(use_tc_tiling_on_sc=False),
        out_type=[
            jax.ShapeDtypeStruct((n, d), jnp.float32),
            jax.ShapeDtypeStruct((n,), jnp.float32),
        ],
        scratch_types=[
            pltpu.VMEM((per_w,), jnp.int32),
            pltpu.VMEM((ch, d), jnp.float32),
            pltpu.VMEM((per_w,), jnp.float32),
            pltpu.SemaphoreType.DMA,
        ],
    )
    def k(x_hbm, emb_hbm, lin_hbm, emb_out, lin_out,
          idx_v, rows_v, lin_v, sem):
        wid = lax.axis_index("s") * 2 + lax.axis_index("c")
        base = wid * per_w
        pltpu.sync_copy(x_hbm.at[pl.ds(base, per_w)], idx_v)
        lin_dma = pltpu.async_copy(lin_hbm.at[idx_v], lin_v, sem)

        for i in range(n_ch):
            pltpu.async_copy(
                emb_hbm.at[idx_v.at[pl.ds(i * ch, ch)]], rows_v, sem
            ).wait()
            pltpu.sync_copy(rows_v, emb_out.at[pl.ds(base + i * ch, ch)])

        lin_dma.wait()
        pltpu.sync_copy(lin_v, lin_out.at[pl.ds(base, per_w)])

    return k(x_flat, emb_table, lin2)


def _fm_l1_body(e_ref, lin_ref, w1_ref, b1_ref, s_ref, h1_ref, fm_ref):
    e = e_ref[...]
    fm1 = jnp.sum(lin_ref[...], axis=1, keepdims=True)
    s = jnp.dot(e, s_ref[...], preferred_element_type=jnp.float32)
    q = jnp.sum(e * e, axis=1, keepdims=True)
    fm2 = 0.5 * (jnp.sum(s * s, axis=1, keepdims=True) - q)
    fm_ref[...] = fm1 + fm2
    h1_ref[...] = (
        jnp.dot(e, w1_ref[...], preferred_element_type=jnp.float32)
        + b1_ref[...]
    )


def _bn(h, g, be):
    mean = jnp.mean(h, axis=0, keepdims=True)
    var = jnp.mean((h - mean) ** 2, axis=0, keepdims=True)
    return (h - mean) * lax.rsqrt(var + EPS) * g + be


def _head_body(h1_ref, fm_ref, w2_ref, b2_ref, w3_ref, b3_ref, w4_ref, b4_ref,
               g1_ref, be1_ref, g2_ref, be2_ref, g3_ref, be3_ref, out_ref):
    h = _bn(h1_ref[...], g1_ref[...], be1_ref[...])
    h = jnp.maximum(h, 0.0)
    h = jnp.dot(h, w2_ref[...], preferred_element_type=jnp.float32) + b2_ref[...]
    h = _bn(h, g2_ref[...], be2_ref[...])
    h = jnp.maximum(h, 0.0)
    h = jnp.dot(h, w3_ref[...], preferred_element_type=jnp.float32) + b3_ref[...]
    h = _bn(h, g3_ref[...], be3_ref[...])
    h = jnp.maximum(h, 0.0)
    deep = jnp.dot(h, w4_ref[...], preferred_element_type=jnp.float32) + b4_ref[...]
    out_ref[...] = jax.nn.sigmoid(fm_ref[...] + deep)


def kernel(x, emb_table, lin_table, W1, b1, W2, b2, W3, b3, W4, b4,
           g1, be1, g2, be2, g3, be3):
    batch, nf = x.shape
    d = emb_table.shape[1]
    f = nf * d
    h1_dim = W1.shape[1]

    emb_rows, lin_rows = _sc_gather(x.reshape(-1), emb_table, lin_table)
    e = emb_rows.reshape(batch, f)
    lin = lin_rows.reshape(batch, nf)

    s_mat = jnp.tile(jnp.eye(d, dtype=jnp.float32), (nf, 1))

    bb = 512
    grid = batch // bb
    h1, fm = pl.pallas_call(
        _fm_l1_body,
        grid=(grid,),
        in_specs=[
            pl.BlockSpec((bb, f), lambda i: (i, 0)),
            pl.BlockSpec((bb, nf), lambda i: (i, 0)),
            pl.BlockSpec(W1.shape, lambda i: (0, 0)),
            pl.BlockSpec((1, h1_dim), lambda i: (0, 0)),
            pl.BlockSpec((f, d), lambda i: (0, 0)),
        ],
        out_specs=[
            pl.BlockSpec((bb, h1_dim), lambda i: (i, 0)),
            pl.BlockSpec((bb, 1), lambda i: (i, 0)),
        ],
        out_shape=[
            jax.ShapeDtypeStruct((batch, h1_dim), jnp.float32),
            jax.ShapeDtypeStruct((batch, 1), jnp.float32),
        ],
    )(e, lin, W1, b1.reshape(1, -1), s_mat)

    row = lambda v: v.reshape(1, -1)
    out = pl.pallas_call(
        _head_body,
        out_shape=jax.ShapeDtypeStruct((batch, 1), jnp.float32),
    )(h1, fm, W2, row(b2), W3, row(b3), W4, row(b4),
      row(g1), row(be1), row(g2), row(be2), row(g3), row(be3))
    return out
